# Initial kernel scaffold; baseline (speedup 1.0000x reference)
import jax
import jax.numpy as jnp
from jax.experimental import pallas as pl


def kernel(x, edge_index, params):
    raise NotImplementedError("write your pallas kernel here")


# speedup vs baseline: 3.1911x; 3.1911x over previous
import functools

import jax
import jax.numpy as jnp
from jax import lax
from jax.experimental import pallas as pl
from jax.experimental.pallas import tpu as pltpu
from jax.experimental.pallas import tpu_sc as plsc

NC = 2
NS = 16
NW = NC * NS
CHUNK = 128


def _make_sc_segment_sum(n_pad, d, chunks):
    mesh = plsc.VectorSubcoreMesh(core_axis_name="c", subcore_axis_name="s")
    rows_per_tile = n_pad // NS

    @functools.partial(
        pl.kernel,
        out_type=jax.ShapeDtypeStruct((NC, n_pad, d), jnp.float32),
        mesh=mesh,
        scratch_types=[
            pltpu.VMEM((chunks, CHUNK), jnp.int32),
            pltpu.VMEM((chunks, CHUNK), jnp.int32),
            pltpu.VMEM((CHUNK, d), jnp.float32),
            pltpu.VMEM_SHARED((n_pad, d), jnp.float32),
            pltpu.SemaphoreType.DMA,
        ],
    )
    def k(h_hbm, zeros_hbm, src_hbm, dst_hbm, out_hbm,
          src_v, dst_v, rows_v, acc, sem):
        cid = lax.axis_index("c")
        sid = lax.axis_index("s")
        wid = cid * NS + sid
        base = sid * rows_per_tile
        @pl.when(cid == 0)
        def _():
            pltpu.sync_copy(h_hbm.at[pl.ds(base, rows_per_tile)],
                            acc.at[pl.ds(base, rows_per_tile)])

        @pl.when(cid != 0)
        def _():
            pltpu.sync_copy(zeros_hbm.at[pl.ds(base, rows_per_tile)],
                            acc.at[pl.ds(base, rows_per_tile)])

        pltpu.sync_copy(src_hbm.at[wid], src_v)
        pltpu.sync_copy(dst_hbm.at[wid], dst_v)
        plsc.subcore_barrier()

        def body(j, carry):
            pltpu.async_copy(h_hbm.at[src_v.at[j]], rows_v, sem).wait()
            pltpu.sync_copy(rows_v, acc.at[dst_v.at[j]], add=True)
            return carry

        lax.fori_loop(0, chunks, body, 0)
        plsc.subcore_barrier()
        pltpu.sync_copy(acc.at[pl.ds(base, rows_per_tile)],
                        out_hbm.at[cid].at[pl.ds(base, rows_per_tile)])

    return k


def _make_mlp(n_pad, d, bm, final):
    inv_std = float((1.0 + 1e-5) ** -0.5)

    def body(p_ref, w1_ref, b1_ref, w2_ref, b2_ref, g_ref, be_ref, o_ref):
        z = p_ref[0] + p_ref[1]
        y = jnp.dot(z, w1_ref[...], preferred_element_type=jnp.float32)
        y = jnp.maximum(y + b1_ref[...], 0.0)
        y = jnp.dot(y, w2_ref[...], preferred_element_type=jnp.float32)
        y = y + b2_ref[...]
        y = y * (g_ref[...] * inv_std) + be_ref[...]
        if not final:
            y = jnp.maximum(y, 0.0)
        o_ref[...] = y

    grid = n_pad // bm
    full = lambda i: (0, 0)
    return pl.pallas_call(
        body,
        grid=(grid,),
        in_specs=[
            pl.BlockSpec((NC, bm, d), lambda i: (0, i, 0)),
            pl.BlockSpec((d, d), full),
            pl.BlockSpec((d,), lambda i: (0,)),
            pl.BlockSpec((d, d), full),
            pl.BlockSpec((d,), lambda i: (0,)),
            pl.BlockSpec((d,), lambda i: (0,)),
            pl.BlockSpec((d,), lambda i: (0,)),
        ],
        out_specs=pl.BlockSpec((bm, d), lambda i: (i, 0)),
        out_shape=jax.ShapeDtypeStruct((n_pad, d), jnp.float32),
    )


def kernel(x, edge_index, params):
    n, d = x.shape
    e = edge_index.shape[1]
    n_pad = ((n + 1 + 511) // 512) * 512
    chunks = -(-e // (NW * CHUNK))
    if chunks % 2:
        chunks += 1
    e_pad = NW * chunks * CHUNK

    pad_idx = jnp.full((e_pad - e,), n, dtype=jnp.int32)
    src = jnp.concatenate([edge_index[0], pad_idx]).reshape(NW, chunks, CHUNK)
    dst = jnp.concatenate([edge_index[1], pad_idx]).reshape(NW, chunks, CHUNK)

    h = jnp.concatenate([x, jnp.zeros((n_pad - n, d), jnp.float32)])
    zeros = jnp.zeros((n_pad, d), jnp.float32)

    seg = _make_sc_segment_sum(n_pad, d, chunks)
    for i, (w1, b1, w2, b2, gamma, beta) in enumerate(params):
        p = seg(h, zeros, src, dst)
        mlp = _make_mlp(n_pad, d, 1024, final=(i == len(params) - 1))
        h = mlp(p, w1, b1, w2, b2, gamma, beta)
    return h[:n]


# speedup vs baseline: 11.1588x; 3.4969x over previous
import functools

import jax
import jax.numpy as jnp
from jax import lax
from jax.experimental import pallas as pl
from jax.experimental.pallas import tpu as pltpu
from jax.experimental.pallas import tpu_sc as plsc

NC = 2
NS = 16
NW = NC * NS
CHUNK = 128


def _make_sc_segment_sum(n_pad, d, chunks):
    mesh = plsc.VectorSubcoreMesh(core_axis_name="c", subcore_axis_name="s")
    rows_per_tile = n_pad // NS

    @functools.partial(
        pl.kernel,
        out_type=jax.ShapeDtypeStruct((NC, n_pad, d), jnp.float32),
        mesh=mesh,
        scratch_types=[
            pltpu.VMEM((chunks // 2, CHUNK), jnp.int32),
            pltpu.VMEM((chunks // 2, CHUNK), jnp.int32),
            pltpu.VMEM((CHUNK, d), jnp.float32),
            pltpu.VMEM((CHUNK, d), jnp.float32),
            pltpu.VMEM_SHARED((n_pad, d), jnp.float32),
            pltpu.SemaphoreType.DMA,
            pltpu.SemaphoreType.DMA,
        ],
    )
    def k(h_hbm, zeros_hbm, src_hbm, dst_hbm, out_hbm,
          src_v, dst_v, buf_a, buf_b, acc, sem_a, sem_b):
        cid = lax.axis_index("c")
        sid = lax.axis_index("s")
        wid = cid * NS + sid
        base = sid * rows_per_tile
        @pl.when(cid == 0)
        def _():
            pltpu.sync_copy(h_hbm.at[pl.ds(base, rows_per_tile)],
                            acc.at[pl.ds(base, rows_per_tile)])

        @pl.when(cid != 0)
        def _():
            pltpu.sync_copy(zeros_hbm.at[pl.ds(base, rows_per_tile)],
                            acc.at[pl.ds(base, rows_per_tile)])

        plsc.subcore_barrier()

        def g_start(j, buf, sem):
            pltpu.async_copy(h_hbm.at[src_v.at[j]], buf, sem)

        def g_wait(j, buf, sem):
            pltpu.make_async_copy(h_hbm.at[src_v.at[j]], buf, sem).wait()

        def scat(j, buf):
            pltpu.sync_copy(buf, acc.at[dst_v.at[j]], add=True)

        g = chunks // 2
        for phase in range(2):
            pltpu.sync_copy(src_hbm.at[wid].at[pl.ds(phase * g, g)], src_v)
            pltpu.sync_copy(dst_hbm.at[wid].at[pl.ds(phase * g, g)], dst_v)

            g_start(0, buf_a, sem_a)

            def body(jj, carry):
                j0 = 2 * jj
                g_start(j0 + 1, buf_b, sem_b)
                g_wait(j0, buf_a, sem_a)
                scat(j0, buf_a)

                @pl.when(jj < g // 2 - 1)
                def _():
                    g_start(j0 + 2, buf_a, sem_a)

                g_wait(j0 + 1, buf_b, sem_b)
                scat(j0 + 1, buf_b)
                return carry

            lax.fori_loop(0, g // 2, body, 0)
        plsc.subcore_barrier()
        pltpu.sync_copy(acc.at[pl.ds(base, rows_per_tile)],
                        out_hbm.at[cid].at[pl.ds(base, rows_per_tile)])

    return k


def _make_mlp(n_pad, d, bm, final):
    inv_std = float((1.0 + 1e-5) ** -0.5)

    def body(p_ref, w1_ref, b1_ref, w2_ref, b2_ref, g_ref, be_ref, o_ref):
        z = p_ref[0] + p_ref[1]
        y = jnp.dot(z, w1_ref[...], preferred_element_type=jnp.float32)
        y = jnp.maximum(y + b1_ref[...], 0.0)
        y = jnp.dot(y, w2_ref[...], preferred_element_type=jnp.float32)
        y = y + b2_ref[...]
        y = y * (g_ref[...] * inv_std) + be_ref[...]
        if not final:
            y = jnp.maximum(y, 0.0)
        o_ref[...] = y

    grid = n_pad // bm
    full = lambda i: (0, 0)
    return pl.pallas_call(
        body,
        grid=(grid,),
        in_specs=[
            pl.BlockSpec((NC, bm, d), lambda i: (0, i, 0)),
            pl.BlockSpec((d, d), full),
            pl.BlockSpec((d,), lambda i: (0,)),
            pl.BlockSpec((d, d), full),
            pl.BlockSpec((d,), lambda i: (0,)),
            pl.BlockSpec((d,), lambda i: (0,)),
            pl.BlockSpec((d,), lambda i: (0,)),
        ],
        out_specs=pl.BlockSpec((bm, d), lambda i: (i, 0)),
        out_shape=jax.ShapeDtypeStruct((n_pad, d), jnp.float32),
    )


def kernel(x, edge_index, params):
    n, d = x.shape
    e = edge_index.shape[1]
    n_pad = ((n + 1 + 511) // 512) * 512
    chunks = -(-e // (NW * CHUNK))
    chunks = ((chunks + 3) // 4) * 4
    e_pad = NW * chunks * CHUNK

    pad_idx = n + (jnp.arange(e_pad - e, dtype=jnp.int32) % (n_pad - n))
    src = jnp.concatenate([edge_index[0], pad_idx]).reshape(NW, chunks, CHUNK)
    dst = jnp.concatenate([edge_index[1], pad_idx]).reshape(NW, chunks, CHUNK)

    h = jnp.concatenate([x, jnp.zeros((n_pad - n, d), jnp.float32)])
    zeros = jnp.zeros((n_pad, d), jnp.float32)

    seg = _make_sc_segment_sum(n_pad, d, chunks)
    for i, (w1, b1, w2, b2, gamma, beta) in enumerate(params):
        p = seg(h, zeros, src, dst)
        mlp = _make_mlp(n_pad, d, 1024, final=(i == len(params) - 1))
        h = mlp(p, w1, b1, w2, b2, gamma, beta)
    return h[:n]


# speedup vs baseline: 11.1803x; 1.0019x over previous
import functools

import jax
import jax.numpy as jnp
from jax import lax
from jax.experimental import pallas as pl
from jax.experimental.pallas import tpu as pltpu
from jax.experimental.pallas import tpu_sc as plsc

NC = 2
NS = 16
NW = NC * NS
CHUNK = 96
NBUF = 3
PHASES = 4


def _make_sc_segment_sum(n_pad, d, pc):
    mesh = plsc.VectorSubcoreMesh(core_axis_name="c", subcore_axis_name="s")
    rows_per_tile = n_pad // NS

    @functools.partial(
        pl.kernel,
        out_type=jax.ShapeDtypeStruct((NC, n_pad, d), jnp.float32),
        mesh=mesh,
        scratch_types=[
            pltpu.VMEM((pc, CHUNK), jnp.int32),
            pltpu.VMEM((pc, CHUNK), jnp.int32),
            [pltpu.VMEM((CHUNK, d), jnp.float32) for _ in range(NBUF)],
            [pltpu.SemaphoreType.DMA for _ in range(NBUF)],
            [pltpu.SemaphoreType.DMA for _ in range(NBUF)],
            pltpu.VMEM_SHARED((n_pad, d), jnp.float32),
        ],
    )
    def k(h_hbm, zeros_hbm, src_hbm, dst_hbm, out_hbm,
          src_v, dst_v, bufs, gsems, ssems, acc):
        cid = lax.axis_index("c")
        sid = lax.axis_index("s")
        wid = cid * NS + sid
        base = sid * rows_per_tile
        @pl.when(cid == 0)
        def _():
            pltpu.sync_copy(h_hbm.at[pl.ds(base, rows_per_tile)],
                            acc.at[pl.ds(base, rows_per_tile)])

        @pl.when(cid != 0)
        def _():
            pltpu.sync_copy(zeros_hbm.at[pl.ds(base, rows_per_tile)],
                            acc.at[pl.ds(base, rows_per_tile)])

        plsc.subcore_barrier()

        def g_start(j, b):
            pltpu.async_copy(h_hbm.at[src_v.at[j]], bufs[b], gsems[b])

        def g_wait(j, b):
            pltpu.make_async_copy(h_hbm.at[src_v.at[j]], bufs[b],
                                  gsems[b]).wait()

        def s_start(j, b):
            pltpu.async_copy(bufs[b], acc.at[dst_v.at[j]], ssems[b], add=True)

        def s_wait(j, b):
            pltpu.make_async_copy(bufs[b], acc.at[dst_v.at[j]],
                                  ssems[b]).wait()

        n_it = pc // 3
        for ph in range(PHASES):
            pltpu.sync_copy(src_hbm.at[wid].at[ph], src_v)
            pltpu.sync_copy(dst_hbm.at[wid].at[ph], dst_v)
            g_start(0, 0)
            g_start(1, 1)

            def body(jj, carry):
                j0 = 3 * jj
                for t in range(3):
                    j = j0 + t
                    g_wait(j, t)
                    s_start(j, t)
                    nxt = (t + 2) % 3
                    if t == 0:
                        @pl.when(jj > 0)
                        def _(j=j, nxt=nxt):
                            s_wait(j - 1, nxt)
                        g_start(j + 2, nxt)
                    else:
                        @pl.when(jj < n_it - 1)
                        def _(j=j, nxt=nxt):
                            s_wait(j - 1, nxt)
                            g_start(j + 2, nxt)
                return carry

            lax.fori_loop(0, n_it, body, 0)
            for t in range(3):
                j = pc - 3 + t
                s_wait(j, j % 3)
        plsc.subcore_barrier()
        pltpu.sync_copy(acc.at[pl.ds(base, rows_per_tile)],
                        out_hbm.at[cid].at[pl.ds(base, rows_per_tile)])

    return k


def _make_mlp(n_pad, d, bm, final):
    inv_std = float((1.0 + 1e-5) ** -0.5)

    def body(p_ref, w1_ref, b1_ref, w2_ref, b2_ref, g_ref, be_ref, o_ref):
        z = p_ref[0] + p_ref[1]
        y = jnp.dot(z, w1_ref[...], preferred_element_type=jnp.float32)
        y = jnp.maximum(y + b1_ref[...], 0.0)
        y = jnp.dot(y, w2_ref[...], preferred_element_type=jnp.float32)
        y = y + b2_ref[...]
        y = y * (g_ref[...] * inv_std) + be_ref[...]
        if not final:
            y = jnp.maximum(y, 0.0)
        o_ref[...] = y

    grid = n_pad // bm
    full = lambda i: (0, 0)
    return pl.pallas_call(
        body,
        grid=(grid,),
        in_specs=[
            pl.BlockSpec((NC, bm, d), lambda i: (0, i, 0)),
            pl.BlockSpec((d, d), full),
            pl.BlockSpec((d,), lambda i: (0,)),
            pl.BlockSpec((d, d), full),
            pl.BlockSpec((d,), lambda i: (0,)),
            pl.BlockSpec((d,), lambda i: (0,)),
            pl.BlockSpec((d,), lambda i: (0,)),
        ],
        out_specs=pl.BlockSpec((bm, d), lambda i: (i, 0)),
        out_shape=jax.ShapeDtypeStruct((n_pad, d), jnp.float32),
    )


def kernel(x, edge_index, params):
    n, d = x.shape
    e = edge_index.shape[1]
    n_pad = ((n + 1 + 127) // 128) * 128
    chunks = -(-e // (NW * CHUNK))
    pc = -(-chunks // PHASES)
    pc = ((pc + 2) // 3) * 3
    chunks = PHASES * pc
    e_pad = NW * chunks * CHUNK

    bm = n_pad
    while bm > 4096 or bm % 8:
        bm //= 2

    pad_idx = n + (jnp.arange(e_pad - e, dtype=jnp.int32) % (n_pad - n))
    src = jnp.concatenate([edge_index[0], pad_idx]).reshape(NW, PHASES, pc, CHUNK)
    dst = jnp.concatenate([edge_index[1], pad_idx]).reshape(NW, PHASES, pc, CHUNK)

    h = jnp.concatenate([x, jnp.zeros((n_pad - n, d), jnp.float32)])
    zeros = jnp.zeros((n_pad, d), jnp.float32)

    seg = _make_sc_segment_sum(n_pad, d, pc)
    for i, (w1, b1, w2, b2, gamma, beta) in enumerate(params):
        p = seg(h, zeros, src, dst)
        mlp = _make_mlp(n_pad, d, bm, final=(i == len(params) - 1))
        h = mlp(p, w1, b1, w2, b2, gamma, beta)
    return h[:n]
